# baseline (speedup 1.0000x reference)
import functools

import jax
import jax.numpy as jnp
from jax import lax
from jax.experimental import pallas as pl
from jax.experimental.pallas import tpu as pltpu
from jax.experimental.pallas import tpu_sc as plsc

B = 16384
F = 26
V = 1000
D = 128
L = 16
NC, NS = 2, 16
NW = NC * NS
BPW = B // NW
TPW = BPW * F
NSP = 8
NH = F - NSP
KE = 4
SROWS = KE * NSP
HROWS = KE * NH
NBLK = BPW // KE
NBUF = 2
NCH = D // L


def _lane_sum(v):
    lane = lax.iota(jnp.int32, L)
    dnums = lax.GatherDimensionNumbers(
        offset_dims=(), collapsed_slice_dims=(0,), start_index_map=(0,))
    for sh in (1, 2, 4, 8):
        perm = lane ^ sh
        v = v + lax.gather(v, perm[:, None], dnums, (1,),
                           mode=lax.GatherScatterMode.PROMISE_IN_BOUNDS)
    return v


def _rsqrt_vec(v):
    y = plsc.bitcast(v, jnp.int32)
    y = jnp.int32(0x5F3759DF) - (y >> 1)
    x = plsc.bitcast(y, jnp.float32)
    half = v * 0.5
    for _ in range(2):
        x = x * (1.5 - half * x * x)
    return x


def _body(tables_hbm, tokens_hbm, scale_hbm, bias_hbm, out_hbm,
          sp_tab, tok_v, isp_v, ihb_v, rows_sp, rows_hb, outb_v,
          scale_v, bias_v, stage_sem, gsems, hsems, ssems):
    cid = lax.axis_index("c")
    sid = lax.axis_index("s")
    wid = sid * NC + cid
    tok_base = wid * TPW
    row_base = wid * BPW

    spr_a, spr_b = 504, NSP * V - 15 * 504

    @pl.when(sid < NS - 1)
    def _stage_a():
        pltpu.make_async_copy(
            tables_hbm.at[pl.ds(sid * spr_a, spr_a)],
            sp_tab.at[pl.ds(sid * spr_a, spr_a)], stage_sem).start()

    @pl.when(sid == NS - 1)
    def _stage_b():
        pltpu.make_async_copy(
            tables_hbm.at[pl.ds(15 * spr_a, spr_b)],
            sp_tab.at[pl.ds(15 * spr_a, spr_b)], stage_sem).start()
    pltpu.sync_copy(tokens_hbm.at[pl.ds(tok_base, TPW)], tok_v)
    pltpu.sync_copy(scale_hbm, scale_v)
    pltpu.sync_copy(bias_hbm, bias_v)

    lane = lax.iota(jnp.int32, L)

    @pl.loop(0, BPW * NSP // L)
    def _isp_loop(i):
        n = i * L + lane
        e = n >> 3
        k = n & (NSP - 1)
        t = plsc.load_gather(tok_v, [e * F + k])
        t = jnp.minimum(jnp.maximum(t, 0), V - 1)
        isp_v[pl.ds(i * L, L)] = t + k * V

    @pl.loop(0, BPW * NH // L)
    def _ihb_loop(i):
        n = i * L + lane
        e = n // NH
        k = n % NH
        t = plsc.load_gather(tok_v, [e * F + NSP + k])
        t = jnp.minimum(jnp.maximum(t, 0), V - 1)
        ihb_v[pl.ds(i * L, L)] = t + (k + NSP) * V

    @pl.when(sid < NS - 1)
    def _stage_wait_a():
        pltpu.make_async_copy(
            tables_hbm.at[pl.ds(sid * spr_a, spr_a)],
            sp_tab.at[pl.ds(sid * spr_a, spr_a)], stage_sem).wait()

    @pl.when(sid == NS - 1)
    def _stage_wait_b():
        pltpu.make_async_copy(
            tables_hbm.at[pl.ds(15 * spr_a, spr_b)],
            sp_tab.at[pl.ds(15 * spr_a, spr_b)], stage_sem).wait()

    plsc.subcore_barrier()

    sc = [scale_v[pl.ds(c * L, L)] for c in range(NCH)]
    bs = [bias_v[pl.ds(c * L, L)] for c in range(NCH)]

    def gathers(j, p):
        return [
            pltpu.make_async_copy(
                sp_tab.at[isp_v.at[pl.ds(j * SROWS, SROWS)]],
                rows_sp[p], gsems[p]),
            pltpu.make_async_copy(
                tables_hbm.at[ihb_v.at[pl.ds(j * HROWS, HROWS)]],
                rows_hb[p], hsems[p]),
        ]

    def store(j, p):
        return pltpu.make_async_copy(
            outb_v[p], out_hbm.at[pl.ds(row_base + j * KE, KE)], ssems[p])

    for p in range(NBUF):
        for d in gathers(p, p):
            d.start()

    def compute_block(p):
        @pl.loop(0, KE)
        def _elem(b):
            rs = b * NSP
            rh = b * NH
            acc = [rows_sp[p][rs, pl.ds(c * L, L)] for c in range(NCH)]
            for f in range(1, NSP):
                for c in range(NCH):
                    acc[c] = acc[c] + rows_sp[p][rs + f, pl.ds(c * L, L)]
            for f in range(NH):
                for c in range(NCH):
                    acc[c] = acc[c] + rows_hb[p][rh + f, pl.ds(c * L, L)]
            s = acc[0]
            sq = acc[0] * acc[0]
            for c in range(1, NCH):
                s = s + acc[c]
                sq = sq + acc[c] * acc[c]
            meanv = _lane_sum(s) * (1.0 / D)
            varv = _lane_sum(sq) * (1.0 / D) - meanv * meanv
            rinv = _rsqrt_vec(varv + 1e-12)
            for c in range(NCH):
                outb_v[p][b, pl.ds(c * L, L)] = (
                    (acc[c] - meanv) * rinv * sc[c] + bs[c])

    @pl.loop(0, NBLK // NBUF)
    def _outer(t):
        for p in range(NBUF):
            j = t * NBUF + p
            for d in gathers(j, p):
                d.wait()

            @pl.when(j >= NBUF)
            def _wait_store():
                store(0, p).wait()

            compute_block(p)
            store(j, p).start()

            @pl.when(t < NBLK // NBUF - 1)
            def _next_gather():
                for d in gathers(j + NBUF, p):
                    d.start()

    for p in range(NBUF):
        store(0, p).wait()


_emb_kernel = functools.partial(
    pl.kernel,
    compiler_params=pltpu.CompilerParams(needs_layout_passes=False),
    out_type=jax.ShapeDtypeStruct((B, D), jnp.float32),
    mesh=plsc.VectorSubcoreMesh(
        core_axis_name="c", subcore_axis_name="s",
        num_cores=NC, num_subcores=NS),
    scratch_types=dict(
        sp_tab=pltpu.VMEM_SHARED((NSP * V, D), jnp.float32),
        tok_v=pltpu.VMEM((TPW,), jnp.int32),
        isp_v=pltpu.VMEM((BPW * NSP,), jnp.int32),
        ihb_v=pltpu.VMEM((BPW * NH,), jnp.int32),
        rows_sp=[pltpu.VMEM((SROWS, D), jnp.float32) for _ in range(NBUF)],
        rows_hb=[pltpu.VMEM((HROWS, D), jnp.float32) for _ in range(NBUF)],
        outb_v=[pltpu.VMEM((KE, D), jnp.float32) for _ in range(NBUF)],
        scale_v=pltpu.VMEM((D,), jnp.float32),
        bias_v=pltpu.VMEM((D,), jnp.float32),
        stage_sem=pltpu.SemaphoreType.DMA,
        gsems=[pltpu.SemaphoreType.DMA for _ in range(NBUF)],
        hsems=[pltpu.SemaphoreType.DMA for _ in range(NBUF)],
        ssems=[pltpu.SemaphoreType.DMA for _ in range(NBUF)],
    ),
)(_body)


def kernel(tokens, eval, tables, ln_scale, ln_bias):
    del eval
    tokens_flat = tokens.reshape(-1).astype(jnp.int32)
    tables_flat = tables.reshape(F * V, D).astype(jnp.float32)
    return _emb_kernel(tables_flat, tokens_flat,
                       ln_scale.astype(jnp.float32),
                       ln_bias.astype(jnp.float32))
